# baseline (device time: 82886 ns/iter reference)
import jax
import jax.numpy as jnp
from jax import lax
from jax.experimental import pallas as pl
from jax.experimental.pallas import tpu as pltpu

_CLIP = 5.5
_STEP = _CLIP / 127.0


def kernel(Q, K, V):
    b, sq, h, d = Q.shape
    scale = d ** -0.5

    Kb = jnp.transpose(K, (0, 2, 1, 3)).astype(jnp.bfloat16)
    Vb = jnp.transpose(V, (0, 2, 1, 3)).astype(jnp.bfloat16)

    def quant(x):
        x = x.astype(jnp.float32)
        return jnp.round(jnp.clip(x, -_CLIP, _CLIP) * (1.0 / _STEP)).astype(jnp.int8)

    Kq = quant(Kb)
    Vq = quant(Vb)

    def body(q_blk, k_ref, v_ref, kq_ref, vq_ref, o_blk,
             krecv, vrecv, send_sems, recv_sems):
        bi = pl.program_id(0)

        partner = (lax.axis_index("x"), 1 - lax.axis_index("y"),
                   lax.axis_index("z"))
        barrier = pltpu.get_barrier_semaphore()

        def chunk_rdma(tensor_idx, src, dst, bb):
            return pltpu.make_async_remote_copy(
                src_ref=src.at[bb], dst_ref=dst.at[bb],
                send_sem=send_sems.at[tensor_idx, bb],
                recv_sem=recv_sems.at[tensor_idx, bb],
                device_id=partner, device_id_type=pl.DeviceIdType.MESH,
            )

        @pl.when(bi == 0)
        def _entry():
            pl.semaphore_signal(barrier, inc=1, device_id=partner,
                                device_id_type=pl.DeviceIdType.MESH)
            pl.semaphore_wait(barrier, 1)
            for bb in range(b):
                chunk_rdma(0, kq_ref, krecv, bb).start()
                chunk_rdma(1, vq_ref, vrecv, bb).start()

        chunk_rdma(0, kq_ref, krecv, bi).wait_recv()
        chunk_rdma(1, vq_ref, vrecv, bi).wait_recv()

        for hh in range(h):
            q = q_blk[0, :, hh, :].astype(jnp.bfloat16)
            q1 = q * jnp.bfloat16(scale)
            q2 = q * jnp.bfloat16(scale * _STEP)
            k1 = k_ref[bi, hh]
            k2 = krecv[bi, hh].astype(jnp.bfloat16)
            s1 = jax.lax.dot_general(q1, k1, (((1,), (1,)), ((), ())),
                                     preferred_element_type=jnp.float32)
            s2 = jax.lax.dot_general(q2, k2, (((1,), (1,)), ((), ())),
                                     preferred_element_type=jnp.float32)
            e1 = jnp.exp(s1.astype(jnp.bfloat16))
            e2 = jnp.exp(s2.astype(jnp.bfloat16))
            l = (jnp.sum(e1, axis=1, dtype=jnp.float32, keepdims=True)
                 + jnp.sum(e2, axis=1, dtype=jnp.float32, keepdims=True))
            o1 = jax.lax.dot_general(e1, v_ref[bi, hh], (((1,), (0,)), ((), ())),
                                     preferred_element_type=jnp.float32)
            o2 = jax.lax.dot_general(e2, vrecv[bi, hh].astype(jnp.bfloat16),
                                     (((1,), (0,)), ((), ())),
                                     preferred_element_type=jnp.float32)
            o_blk[0, :, hh, :] = (
                (o1 + o2 * jnp.float32(_STEP)) / l).astype(jnp.bfloat16)

        @pl.when(bi == b - 1)
        def _exit():
            for bb in range(b):
                chunk_rdma(0, kq_ref, krecv, bb).wait_send()
                chunk_rdma(1, vq_ref, vrecv, bb).wait_send()
            pl.semaphore_signal(barrier, inc=1, device_id=partner,
                                device_id_type=pl.DeviceIdType.MESH)
            pl.semaphore_wait(barrier, 1)

    out = pl.pallas_call(
        body,
        grid=(b,),
        in_specs=[
            pl.BlockSpec((1, sq, h, d), lambda i: (i, 0, 0, 0)),
            pl.BlockSpec(memory_space=pltpu.VMEM),
            pl.BlockSpec(memory_space=pltpu.VMEM),
            pl.BlockSpec(memory_space=pltpu.VMEM),
            pl.BlockSpec(memory_space=pltpu.VMEM),
        ],
        out_specs=pl.BlockSpec((1, sq, h, d), lambda i: (i, 0, 0, 0)),
        out_shape=jax.ShapeDtypeStruct((b, sq, h, d), jnp.bfloat16),
        scratch_shapes=[
            pltpu.VMEM((b, h, sq, d), jnp.int8),
            pltpu.VMEM((b, h, sq, d), jnp.int8),
            pltpu.SemaphoreType.DMA((2, b)),
            pltpu.SemaphoreType.DMA((2, b)),
        ],
        compiler_params=pltpu.CompilerParams(
            collective_id=0,
            dimension_semantics=("arbitrary",),
        ),
    )(Q, Kb, Vb, Kq, Vq)

    return out


# device time: 70059 ns/iter; 1.1831x vs baseline; 1.1831x over previous
import jax
import jax.numpy as jnp
from jax import lax
from jax.experimental import pallas as pl
from jax.experimental.pallas import tpu as pltpu

_CLIP = 5.5
_STEP = _CLIP / 127.0


def kernel(Q, K, V):
    b, sq, h, d = Q.shape
    scale = d ** -0.5

    Qb = jnp.transpose(Q, (0, 2, 1, 3)).astype(jnp.bfloat16)
    Kb = jnp.transpose(K, (0, 2, 1, 3)).astype(jnp.bfloat16)
    Vb = jnp.transpose(V, (0, 2, 1, 3)).astype(jnp.bfloat16)

    def body(q_blk, k_ref, v_ref, o_blk,
             kq_ref, vq_ref, krecv, vrecv, send_sems, recv_sems):
        bi = pl.program_id(0)
        hi = pl.program_id(1)

        partner = (lax.axis_index("x"), 1 - lax.axis_index("y"),
                   lax.axis_index("z"))
        barrier = pltpu.get_barrier_semaphore()

        def chunk_rdma(tensor_idx, src, dst, bb):
            return pltpu.make_async_remote_copy(
                src_ref=src.at[bb], dst_ref=dst.at[bb],
                send_sem=send_sems.at[tensor_idx, bb],
                recv_sem=recv_sems.at[tensor_idx, bb],
                device_id=partner, device_id_type=pl.DeviceIdType.MESH,
            )

        def quant(x):
            x = x.astype(jnp.float32)
            return jnp.round(
                jnp.clip(x, -_CLIP, _CLIP) * (1.0 / _STEP)).astype(jnp.int8)

        @pl.when(jnp.logical_and(bi == 0, hi == 0))
        def _entry():
            pl.semaphore_signal(barrier, inc=1, device_id=partner,
                                device_id_type=pl.DeviceIdType.MESH)
            pl.semaphore_wait(barrier, 1)
            for bb in range(b):
                kq_ref[bb] = quant(k_ref[bb])
                vq_ref[bb] = quant(v_ref[bb])
                chunk_rdma(0, kq_ref, krecv, bb).start()
                chunk_rdma(1, vq_ref, vrecv, bb).start()

        @pl.when(hi == 0)
        def _wait_chunk():
            chunk_rdma(0, kq_ref, krecv, bi).wait_recv()
            chunk_rdma(1, vq_ref, vrecv, bi).wait_recv()

        q = q_blk[0, 0]
        q1 = q * jnp.bfloat16(scale)
        q2 = q * jnp.bfloat16(scale * _STEP)
        k1 = k_ref[bi, hi]
        k2 = krecv[bi, hi].astype(jnp.bfloat16)
        s1 = jax.lax.dot_general(q1, k1, (((1,), (1,)), ((), ())),
                                 preferred_element_type=jnp.float32)
        s2 = jax.lax.dot_general(q2, k2, (((1,), (1,)), ((), ())),
                                 preferred_element_type=jnp.float32)
        e1 = jnp.exp(s1.astype(jnp.bfloat16))
        e2 = jnp.exp(s2.astype(jnp.bfloat16))
        l = (jnp.sum(e1, axis=1, dtype=jnp.float32, keepdims=True)
             + jnp.sum(e2, axis=1, dtype=jnp.float32, keepdims=True))
        o1 = jax.lax.dot_general(e1, v_ref[bi, hi], (((1,), (0,)), ((), ())),
                                 preferred_element_type=jnp.float32)
        o2 = jax.lax.dot_general(e2, vrecv[bi, hi].astype(jnp.bfloat16),
                                 (((1,), (0,)), ((), ())),
                                 preferred_element_type=jnp.float32)
        o_blk[0, 0] = ((o1 + o2 * jnp.float32(_STEP)) / l).astype(jnp.bfloat16)

        @pl.when(jnp.logical_and(bi == b - 1, hi == h - 1))
        def _exit():
            for bb in range(b):
                chunk_rdma(0, kq_ref, krecv, bb).wait_send()
                chunk_rdma(1, vq_ref, vrecv, bb).wait_send()
            pl.semaphore_signal(barrier, inc=1, device_id=partner,
                                device_id_type=pl.DeviceIdType.MESH)
            pl.semaphore_wait(barrier, 1)

    out = pl.pallas_call(
        body,
        grid=(b, h),
        in_specs=[
            pl.BlockSpec((1, 1, sq, d), lambda i, j: (i, j, 0, 0)),
            pl.BlockSpec(memory_space=pltpu.VMEM),
            pl.BlockSpec(memory_space=pltpu.VMEM),
        ],
        out_specs=pl.BlockSpec((1, 1, sq, d), lambda i, j: (i, j, 0, 0)),
        out_shape=jax.ShapeDtypeStruct((b, h, sq, d), jnp.bfloat16),
        scratch_shapes=[
            pltpu.VMEM((b, h, sq, d), jnp.int8),
            pltpu.VMEM((b, h, sq, d), jnp.int8),
            pltpu.VMEM((b, h, sq, d), jnp.int8),
            pltpu.VMEM((b, h, sq, d), jnp.int8),
            pltpu.SemaphoreType.DMA((2, b)),
            pltpu.SemaphoreType.DMA((2, b)),
        ],
        compiler_params=pltpu.CompilerParams(
            collective_id=0,
            dimension_semantics=("arbitrary", "arbitrary"),
        ),
    )(Qb, Kb, Vb)

    return jnp.transpose(out, (0, 2, 1, 3))


# device time: 67182 ns/iter; 1.2338x vs baseline; 1.0428x over previous
import jax
import jax.numpy as jnp
from jax import lax
from jax.experimental import pallas as pl
from jax.experimental.pallas import tpu as pltpu

_CLIP = 5.5
_STEP = _CLIP / 127.0


def kernel(Q, K, V):
    b, sq, h, d = Q.shape
    scale = d ** -0.5

    Qb = jnp.transpose(Q, (0, 2, 1, 3)).astype(jnp.bfloat16)
    Kb = jnp.transpose(K, (0, 2, 1, 3)).astype(jnp.bfloat16)
    Vb = jnp.transpose(V, (0, 2, 1, 3)).astype(jnp.bfloat16)

    def body(q_blk, k_ref, v_ref, o_blk,
             kq_ref, vq_ref, krecv, vrecv, send_sems, recv_sems):
        bi = pl.program_id(0)

        partner = (lax.axis_index("x"), 1 - lax.axis_index("y"),
                   lax.axis_index("z"))
        barrier = pltpu.get_barrier_semaphore()

        def chunk_rdma(tensor_idx, src, dst, bb):
            return pltpu.make_async_remote_copy(
                src_ref=src.at[bb], dst_ref=dst.at[bb],
                send_sem=send_sems.at[tensor_idx, bb],
                recv_sem=recv_sems.at[tensor_idx, bb],
                device_id=partner, device_id_type=pl.DeviceIdType.MESH,
            )

        def quant(x):
            x = x.astype(jnp.float32)
            return jnp.round(
                jnp.clip(x, -_CLIP, _CLIP) * (1.0 / _STEP)).astype(jnp.int8)

        @pl.when(bi == 0)
        def _entry():
            pl.semaphore_signal(barrier, inc=1, device_id=partner,
                                device_id_type=pl.DeviceIdType.MESH)
            pl.semaphore_wait(barrier, 1)
            for bb in range(b):
                kq_ref[bb] = quant(k_ref[bb])
                vq_ref[bb] = quant(v_ref[bb])
                chunk_rdma(0, kq_ref, krecv, bb).start()
                chunk_rdma(1, vq_ref, vrecv, bb).start()

        chunk_rdma(0, kq_ref, krecv, bi).wait_recv()
        chunk_rdma(1, vq_ref, vrecv, bi).wait_recv()

        for hh in range(h):
            q = q_blk[0, hh]
            q1 = q * jnp.bfloat16(scale)
            q2 = q * jnp.bfloat16(scale * _STEP)
            k1 = k_ref[bi, hh]
            k2 = krecv[bi, hh].astype(jnp.bfloat16)
            s1 = jax.lax.dot_general(q1, k1, (((1,), (1,)), ((), ())),
                                     preferred_element_type=jnp.float32)
            s2 = jax.lax.dot_general(q2, k2, (((1,), (1,)), ((), ())),
                                     preferred_element_type=jnp.float32)
            e1 = jnp.exp(s1.astype(jnp.bfloat16))
            e2 = jnp.exp(s2.astype(jnp.bfloat16))
            l = (jnp.sum(e1, axis=1, dtype=jnp.float32, keepdims=True)
                 + jnp.sum(e2, axis=1, dtype=jnp.float32, keepdims=True))
            o1 = jax.lax.dot_general(e1, v_ref[bi, hh], (((1,), (0,)), ((), ())),
                                     preferred_element_type=jnp.float32)
            o2 = jax.lax.dot_general(e2, vrecv[bi, hh].astype(jnp.bfloat16),
                                     (((1,), (0,)), ((), ())),
                                     preferred_element_type=jnp.float32)
            o_blk[0, hh] = (
                (o1 + o2 * jnp.float32(_STEP)) / l).astype(jnp.bfloat16)

        @pl.when(bi == b - 1)
        def _exit():
            for bb in range(b):
                chunk_rdma(0, kq_ref, krecv, bb).wait_send()
                chunk_rdma(1, vq_ref, vrecv, bb).wait_send()
            pl.semaphore_signal(barrier, inc=1, device_id=partner,
                                device_id_type=pl.DeviceIdType.MESH)
            pl.semaphore_wait(barrier, 1)

    out = pl.pallas_call(
        body,
        grid=(b,),
        in_specs=[
            pl.BlockSpec((1, h, sq, d), lambda i: (i, 0, 0, 0)),
            pl.BlockSpec(memory_space=pltpu.VMEM),
            pl.BlockSpec(memory_space=pltpu.VMEM),
        ],
        out_specs=pl.BlockSpec((1, h, sq, d), lambda i: (i, 0, 0, 0)),
        out_shape=jax.ShapeDtypeStruct((b, h, sq, d), jnp.bfloat16),
        scratch_shapes=[
            pltpu.VMEM((b, h, sq, d), jnp.int8),
            pltpu.VMEM((b, h, sq, d), jnp.int8),
            pltpu.VMEM((b, h, sq, d), jnp.int8),
            pltpu.VMEM((b, h, sq, d), jnp.int8),
            pltpu.SemaphoreType.DMA((2, b)),
            pltpu.SemaphoreType.DMA((2, b)),
        ],
        compiler_params=pltpu.CompilerParams(
            collective_id=0,
            dimension_semantics=("arbitrary",),
        ),
    )(Qb, Kb, Vb)

    return jnp.transpose(out, (0, 2, 1, 3))


# device time: 65374 ns/iter; 1.2679x vs baseline; 1.0277x over previous
import jax
import jax.numpy as jnp
from jax import lax
from jax.experimental import pallas as pl
from jax.experimental.pallas import tpu as pltpu

_CLIP = 5.5
_STEP = _CLIP / 127.0


def kernel(Q, K, V):
    b, sq, h, d = Q.shape
    scale = d ** -0.5

    Qb = jnp.transpose(Q, (0, 2, 1, 3)).astype(jnp.bfloat16)
    Kb = jnp.transpose(K, (0, 2, 1, 3)).astype(jnp.bfloat16)
    Vb = jnp.transpose(V, (0, 2, 1, 3)).astype(jnp.bfloat16)

    def body(q_blk, k_ref, v_ref, o_blk,
             kq_ref, vq_ref, krecv, vrecv, send_sems, recv_sems):
        bi = pl.program_id(0)

        partner = (lax.axis_index("x"), 1 - lax.axis_index("y"),
                   lax.axis_index("z"))
        barrier = pltpu.get_barrier_semaphore()

        hh2 = h // 2

        def chunk_rdma(tensor_idx, src, dst, bb, half):
            sl = slice(half * hh2, (half + 1) * hh2)
            return pltpu.make_async_remote_copy(
                src_ref=src.at[bb, sl], dst_ref=dst.at[bb, sl],
                send_sem=send_sems.at[tensor_idx, bb, half],
                recv_sem=recv_sems.at[tensor_idx, bb, half],
                device_id=partner, device_id_type=pl.DeviceIdType.MESH,
            )

        def quant(x):
            x = x.astype(jnp.float32)
            return jnp.round(
                jnp.clip(x, -_CLIP, _CLIP) * (1.0 / _STEP)).astype(jnp.int8)

        @pl.when(bi == 0)
        def _entry():
            pl.semaphore_signal(barrier, inc=1, device_id=partner,
                                device_id_type=pl.DeviceIdType.MESH)
            pl.semaphore_wait(barrier, 1)
            for bb in range(b):
                kq_ref[bb] = quant(k_ref[bb])
                vq_ref[bb] = quant(v_ref[bb])
                for half in range(2):
                    chunk_rdma(0, kq_ref, krecv, bb, half).start()
                    chunk_rdma(1, vq_ref, vrecv, bb, half).start()

        for hh in range(h):
            if hh % hh2 == 0:
                half = hh // hh2
                chunk_rdma(0, kq_ref, krecv, bi, half).wait_recv()
                chunk_rdma(1, vq_ref, vrecv, bi, half).wait_recv()
            q = q_blk[0, hh]
            q1 = q * jnp.bfloat16(scale)
            q2 = q * jnp.bfloat16(scale * _STEP)
            k1 = k_ref[bi, hh]
            k2 = krecv[bi, hh].astype(jnp.bfloat16)
            s1 = jax.lax.dot_general(q1, k1, (((1,), (1,)), ((), ())),
                                     preferred_element_type=jnp.float32)
            s2 = jax.lax.dot_general(q2, k2, (((1,), (1,)), ((), ())),
                                     preferred_element_type=jnp.float32)
            e1 = jnp.exp(s1.astype(jnp.bfloat16))
            e2 = jnp.exp(s2.astype(jnp.bfloat16))
            l = (jnp.sum(e1, axis=1, dtype=jnp.float32, keepdims=True)
                 + jnp.sum(e2, axis=1, dtype=jnp.float32, keepdims=True))
            o1 = jax.lax.dot_general(e1, v_ref[bi, hh], (((1,), (0,)), ((), ())),
                                     preferred_element_type=jnp.float32)
            o2 = jax.lax.dot_general(e2, vrecv[bi, hh].astype(jnp.bfloat16),
                                     (((1,), (0,)), ((), ())),
                                     preferred_element_type=jnp.float32)
            o_blk[0, hh] = (
                (o1 + o2 * jnp.float32(_STEP)) / l).astype(jnp.bfloat16)

        @pl.when(bi == b - 1)
        def _exit():
            for bb in range(b):
                for half in range(2):
                    chunk_rdma(0, kq_ref, krecv, bb, half).wait_send()
                    chunk_rdma(1, vq_ref, vrecv, bb, half).wait_send()
            pl.semaphore_signal(barrier, inc=1, device_id=partner,
                                device_id_type=pl.DeviceIdType.MESH)
            pl.semaphore_wait(barrier, 1)

    out = pl.pallas_call(
        body,
        grid=(b,),
        in_specs=[
            pl.BlockSpec((1, h, sq, d), lambda i: (i, 0, 0, 0)),
            pl.BlockSpec(memory_space=pltpu.VMEM),
            pl.BlockSpec(memory_space=pltpu.VMEM),
        ],
        out_specs=pl.BlockSpec((1, h, sq, d), lambda i: (i, 0, 0, 0)),
        out_shape=jax.ShapeDtypeStruct((b, h, sq, d), jnp.bfloat16),
        scratch_shapes=[
            pltpu.VMEM((b, h, sq, d), jnp.int8),
            pltpu.VMEM((b, h, sq, d), jnp.int8),
            pltpu.VMEM((b, h, sq, d), jnp.int8),
            pltpu.VMEM((b, h, sq, d), jnp.int8),
            pltpu.SemaphoreType.DMA((2, b, 2)),
            pltpu.SemaphoreType.DMA((2, b, 2)),
        ],
        compiler_params=pltpu.CompilerParams(
            collective_id=0,
            dimension_semantics=("arbitrary",),
        ),
    )(Qb, Kb, Vb)

    return jnp.transpose(out, (0, 2, 1, 3))
